# Initial kernel scaffold; baseline (speedup 1.0000x reference)
import jax
import jax.numpy as jnp
from jax.experimental import pallas as pl


def kernel(edge_index, emb, W1l, b1l, W1r, W2l, b2l, W2r):
    raise NotImplementedError("write your pallas kernel here")


# speedup vs baseline: 4.5564x; 4.5564x over previous
import functools

import jax
import jax.numpy as jnp
from jax import lax
from jax.experimental import pallas as pl
from jax.experimental.pallas import tpu as pltpu
from jax.experimental.pallas import tpu_sc as plsc

N = 10000
E = 640000
D = 128

NC = 2
NS = 16
NW = NC * NS

CH = 128
GCH = 16
NGRP = 10
NCHUNK = NGRP * GCH
EW = NCHUNK * CH
EPAD = NW * EW
NP = 10240
RPS = NP // NS
BN = 1280


def _sc_aggregate(table, srcs, dsts, with_counts):
  out_type = [jax.ShapeDtypeStruct((NC, NP, D), jnp.float32)]
  scratch = [
      pltpu.VMEM((GCH, CH), jnp.int32),
      pltpu.VMEM((GCH, CH), jnp.int32),
      pltpu.VMEM((CH, D), jnp.float32),
      pltpu.VMEM_SHARED((NP, D), jnp.float32),
      pltpu.SemaphoreType.DMA,
  ]
  if with_counts:
    out_type.append(jax.ShapeDtypeStruct((NW, NP), jnp.float32))
    scratch.append(pltpu.VMEM((NP,), jnp.float32))

  mesh = plsc.VectorSubcoreMesh(core_axis_name="c", subcore_axis_name="s")

  def body(table_h, srcs_h, dsts_h, *rest):
    if with_counts:
      p_h, c_h, src_v, dst_v, rows_v, acc_sh, sem, cnt_v = rest
    else:
      p_h, src_v, dst_v, rows_v, acc_sh, sem = rest
    cid = lax.axis_index("c")
    sid = lax.axis_index("s")
    wid = sid * NC + cid

    zeros16 = jnp.zeros((16,), jnp.float32)
    @pl.loop(0, CH)
    def _(i):
      for k in range(D // 16):
        rows_v[i, pl.ds(k * 16, 16)] = zeros16
    if with_counts:
      @pl.loop(0, NP // 16)
      def _(i):
        cnt_v[pl.ds(i * 16, 16)] = zeros16

    for b in range(RPS // CH):
      r0 = sid * RPS + b * CH
      pltpu.sync_copy(rows_v, acc_sh.at[pl.ds(r0, CH)])

    plsc.subcore_barrier()

    ones16 = jnp.ones((16,), jnp.float32)

    @pl.loop(0, NGRP)
    def _(g):
      pltpu.sync_copy(srcs_h.at[wid, pl.ds(g * GCH, GCH)], src_v)
      pltpu.sync_copy(dsts_h.at[wid, pl.ds(g * GCH, GCH)], dst_v)

      @pl.loop(0, GCH)
      def _(j):
        cp = pltpu.async_copy(table_h.at[src_v.at[j]], rows_v, sem)
        if with_counts:
          for k in range(CH // 16):
            idx = dst_v[j, pl.ds(k * 16, 16)]
            plsc.addupdate_scatter(cnt_v, [idx], ones16)
        cp.wait()
        pltpu.sync_copy(rows_v, acc_sh.at[dst_v.at[j]], add=True)

    plsc.subcore_barrier()

    r0 = sid * RPS
    pltpu.sync_copy(acc_sh.at[pl.ds(r0, RPS)], p_h.at[cid, pl.ds(r0, RPS)])
    if with_counts:
      pltpu.sync_copy(cnt_v, c_h.at[wid])

  k = pl.kernel(body, out_type=tuple(out_type), mesh=mesh,
                scratch_types=tuple(scratch),
                compiler_params=pltpu.CompilerParams(
                    needs_layout_passes=False))
  return k(table, srcs, dsts)


def _tc_layer_body(relu, p_ref, c_ref, x_ref, wl_ref, wr_ref, b_ref, o_ref):
  s = p_ref[0] + p_ref[1]
  cnt = jnp.sum(c_ref[...], axis=0)[:, None]
  mean = s / jnp.maximum(cnt, 1.0)
  acc = (jnp.dot(mean, wl_ref[...], preferred_element_type=jnp.float32)
         + jnp.dot(x_ref[...], wr_ref[...], preferred_element_type=jnp.float32)
         + b_ref[...])
  o_ref[...] = jnp.maximum(acc, 0.0) if relu else acc


def _tc_layer(p, c, x, wlt, wrt, b, relu):
  return pl.pallas_call(
      functools.partial(_tc_layer_body, relu),
      grid=(NP // BN,),
      in_specs=[
          pl.BlockSpec((NC, BN, D), lambda i: (0, i, 0)),
          pl.BlockSpec((NW, BN), lambda i: (0, i)),
          pl.BlockSpec((BN, D), lambda i: (i, 0)),
          pl.BlockSpec((D, D), lambda i: (0, 0)),
          pl.BlockSpec((D, D), lambda i: (0, 0)),
          pl.BlockSpec((1, D), lambda i: (0, 0)),
      ],
      out_specs=pl.BlockSpec((BN, D), lambda i: (i, 0)),
      out_shape=jax.ShapeDtypeStruct((NP, D), jnp.float32),
  )(p, c, x, wlt, wrt, b)


def kernel(edge_index, emb, W1l, b1l, W1r, W2l, b2l, W2r):
  src = edge_index[0]
  dst = edge_index[1]
  pad = EPAD - E
  srcs = jnp.concatenate(
      [src, jnp.zeros((pad,), jnp.int32)]).reshape(NW, NCHUNK, CH)
  dsts = jnp.concatenate(
      [dst, jnp.full((pad,), N, jnp.int32)]).reshape(NW, NCHUNK, CH)
  embp = jnp.pad(emb, ((0, NP - N), (0, 0)))

  p1, c = _sc_aggregate(embp, srcs, dsts, with_counts=True)
  h = _tc_layer(p1, c, embp, W1l.T, W1r.T, b1l.reshape(1, D), relu=True)
  (p2,) = _sc_aggregate(h, srcs, dsts, with_counts=False)
  out = _tc_layer(p2, c, h, W2l.T, W2r.T, b2l.reshape(1, D), relu=False)
  return out[:N]


# speedup vs baseline: 4.5617x; 1.0012x over previous
import functools

import jax
import jax.numpy as jnp
from jax import lax
from jax.experimental import pallas as pl
from jax.experimental.pallas import tpu as pltpu
from jax.experimental.pallas import tpu_sc as plsc

N = 10000
E = 640000
D = 128

NC = 2
NS = 16
NW = NC * NS

CH = 128
GCH = 16
NGRP = 10
NCHUNK = NGRP * GCH
EW = NCHUNK * CH
EPAD = NW * EW
NP = 10240
RPS = NP // NS
BN = 1280


def _sc_aggregate(table, srcs, dsts, with_counts):
  out_type = [jax.ShapeDtypeStruct((NC, NP, D), jnp.float32)]
  scratch = [
      pltpu.VMEM((GCH, CH), jnp.int32),
      pltpu.VMEM((GCH, CH), jnp.int32),
      pltpu.VMEM((CH, D), jnp.float32),
      pltpu.VMEM_SHARED((NP, D), jnp.float32),
      pltpu.SemaphoreType.DMA,
  ]
  if with_counts:
    out_type.append(jax.ShapeDtypeStruct((NW, NP), jnp.float32))
    scratch.append(pltpu.VMEM((NP,), jnp.float32))

  mesh = plsc.VectorSubcoreMesh(core_axis_name="c", subcore_axis_name="s")

  def body(table_h, srcs_h, dsts_h, *rest):
    if with_counts:
      p_h, c_h, src_v, dst_v, rows_v, acc_sh, sem, cnt_v = rest
    else:
      p_h, src_v, dst_v, rows_v, acc_sh, sem = rest
    cid = lax.axis_index("c")
    sid = lax.axis_index("s")
    wid = sid * NC + cid

    zeros16 = jnp.zeros((16,), jnp.float32)
    @pl.loop(0, CH)
    def _(i):
      for k in range(D // 16):
        rows_v[i, pl.ds(k * 16, 16)] = zeros16
    if with_counts:
      @pl.loop(0, NP // 16)
      def _(i):
        cnt_v[pl.ds(i * 16, 16)] = zeros16

    for b in range(RPS // CH):
      r0 = sid * RPS + b * CH
      pltpu.sync_copy(rows_v, acc_sh.at[pl.ds(r0, CH)])

    plsc.subcore_barrier()

    ones16 = jnp.ones((16,), jnp.float32)

    @pl.loop(0, NGRP)
    def _(g):
      pltpu.sync_copy(srcs_h.at[wid, pl.ds(g * GCH, GCH)], src_v)
      pltpu.sync_copy(dsts_h.at[wid, pl.ds(g * GCH, GCH)], dst_v)

      @pl.loop(0, GCH)
      def _(j):
        cp = pltpu.async_copy(table_h.at[src_v.at[j]], rows_v, sem)
        if with_counts:
          for k in range(CH // 16):
            idx = dst_v[j, pl.ds(k * 16, 16)]
            plsc.addupdate_scatter(cnt_v, [idx], ones16)
        cp.wait()
        pltpu.sync_copy(rows_v, acc_sh.at[dst_v.at[j]], add=True)

    plsc.subcore_barrier()

    r0 = sid * RPS
    pltpu.sync_copy(acc_sh.at[pl.ds(r0, RPS)], p_h.at[cid, pl.ds(r0, RPS)])
    if with_counts:
      pltpu.sync_copy(cnt_v, c_h.at[wid])

  k = pl.kernel(body, out_type=tuple(out_type), mesh=mesh,
                scratch_types=tuple(scratch),
                compiler_params=pltpu.CompilerParams(
                    needs_layout_passes=False))
  return k(table, srcs, dsts)


def _tc_layer_body(relu, p_ref, c_ref, x_ref, wl_ref, wr_ref, b_ref, o_ref):
  s = p_ref[0] + p_ref[1]
  cnt = jnp.sum(c_ref[...], axis=0)[:, None]
  mean = s / jnp.maximum(cnt, 1.0)
  acc = (jnp.dot(mean, wl_ref[...], preferred_element_type=jnp.float32)
         + jnp.dot(x_ref[...], wr_ref[...], preferred_element_type=jnp.float32)
         + b_ref[...])
  o_ref[...] = jnp.maximum(acc, 0.0) if relu else acc


def _tc_layer(p, c, x, wlt, wrt, b, relu):
  return pl.pallas_call(
      functools.partial(_tc_layer_body, relu),
      grid=(NP // BN,),
      in_specs=[
          pl.BlockSpec((NC, BN, D), lambda i: (0, i, 0)),
          pl.BlockSpec((NW, BN), lambda i: (0, i)),
          pl.BlockSpec((BN, D), lambda i: (i, 0)),
          pl.BlockSpec((D, D), lambda i: (0, 0)),
          pl.BlockSpec((D, D), lambda i: (0, 0)),
          pl.BlockSpec((1, D), lambda i: (0, 0)),
      ],
      out_specs=pl.BlockSpec((BN, D), lambda i: (i, 0)),
      out_shape=jax.ShapeDtypeStruct((NP, D), jnp.float32),
  )(p, c, x, wlt, wrt, b)


def kernel(edge_index, emb, W1l, b1l, W1r, W2l, b2l, W2r):
  src = edge_index[0]
  dst = edge_index[1]
  pad = EPAD - E
  srcs = jnp.concatenate(
      [src, jnp.zeros((pad,), jnp.int32)]).reshape(NW, NCHUNK, CH)
  pad_dst = N + jnp.arange(pad, dtype=jnp.int32) % (NP - N)
  dsts = jnp.concatenate([dst, pad_dst]).reshape(NW, NCHUNK, CH)
  embp = jnp.pad(emb, ((0, NP - N), (0, 0)))

  p1, c = _sc_aggregate(embp, srcs, dsts, with_counts=True)
  h = _tc_layer(p1, c, embp, W1l.T, W1r.T, b1l.reshape(1, D), relu=True)
  (p2,) = _sc_aggregate(h, srcs, dsts, with_counts=False)
  out = _tc_layer(p2, c, h, W2l.T, W2r.T, b2l.reshape(1, D), relu=False)
  return out[:N]


# speedup vs baseline: 4.7021x; 1.0308x over previous
import functools

import jax
import jax.numpy as jnp
from jax import lax
from jax.experimental import pallas as pl
from jax.experimental.pallas import tpu as pltpu
from jax.experimental.pallas import tpu_sc as plsc

N = 10000
E = 640000
D = 128

NC = 2
NS = 16
NW = NC * NS

CH = 128
GCH = 8
TOTCH = 5120
NA = 232
NB = 320 - NA
EPAD = TOTCH * CH
NP = 10240
RPS = NP // NS
BN = 1280


def _sc_aggregate(table, srcs, dsts, with_counts):
  out_type = [jax.ShapeDtypeStruct((NC, NP, D), jnp.float32)]
  scratch = [
      pltpu.VMEM((GCH, CH), jnp.int32),
      pltpu.VMEM((GCH, CH), jnp.int32),
      pltpu.VMEM((CH, D), jnp.float32),
      pltpu.VMEM_SHARED((NP, D), jnp.float32),
      pltpu.SemaphoreType.DMA,
  ]
  if with_counts:
    out_type.append(jax.ShapeDtypeStruct((NW, NP), jnp.float32))
    scratch.append(pltpu.VMEM((NP,), jnp.float32))

  mesh = plsc.VectorSubcoreMesh(core_axis_name="c", subcore_axis_name="s")

  def body(table_h, srcs_h, dsts_h, *rest):
    if with_counts:
      p_h, c_h, src_v, dst_v, rows_v, acc_sh, sem, cnt_v = rest
    else:
      p_h, src_v, dst_v, rows_v, acc_sh, sem = rest
    cid = lax.axis_index("c")
    sid = lax.axis_index("s")
    wid = sid * NC + cid
    start_chunk = jnp.where(cid == 0, sid * NA, NS * NA + sid * NB)
    ngroups = jnp.where(cid == 0, NA // GCH, NB // GCH)

    zeros16 = jnp.zeros((16,), jnp.float32)
    @pl.loop(0, CH)
    def _(i):
      for k in range(D // 16):
        rows_v[i, pl.ds(k * 16, 16)] = zeros16
    if with_counts:
      @pl.loop(0, NP // 16)
      def _(i):
        cnt_v[pl.ds(i * 16, 16)] = zeros16

    for b in range(RPS // CH):
      r0 = sid * RPS + b * CH
      pltpu.sync_copy(rows_v, acc_sh.at[pl.ds(r0, CH)])

    plsc.subcore_barrier()

    ones16 = jnp.ones((16,), jnp.float32)

    @pl.loop(0, ngroups)
    def _(g):
      c0 = start_chunk + g * GCH
      pltpu.sync_copy(srcs_h.at[pl.ds(c0, GCH)], src_v)
      pltpu.sync_copy(dsts_h.at[pl.ds(c0, GCH)], dst_v)

      @pl.loop(0, GCH)
      def _(j):
        cp = pltpu.async_copy(table_h.at[src_v.at[j]], rows_v, sem)
        if with_counts:
          for k in range(CH // 16):
            idx = dst_v[j, pl.ds(k * 16, 16)]
            plsc.addupdate_scatter(cnt_v, [idx], ones16)
        cp.wait()
        pltpu.sync_copy(rows_v, acc_sh.at[dst_v.at[j]], add=True)

    plsc.subcore_barrier()

    r0 = sid * RPS
    pltpu.sync_copy(acc_sh.at[pl.ds(r0, RPS)], p_h.at[cid, pl.ds(r0, RPS)])
    if with_counts:
      pltpu.sync_copy(cnt_v, c_h.at[wid])

  k = pl.kernel(body, out_type=tuple(out_type), mesh=mesh,
                scratch_types=tuple(scratch),
                compiler_params=pltpu.CompilerParams(
                    needs_layout_passes=False))
  return k(table, srcs, dsts)


def _tc_layer_body(relu, p_ref, c_ref, x_ref, wl_ref, wr_ref, b_ref, o_ref):
  s = p_ref[0] + p_ref[1]
  cnt = jnp.sum(c_ref[...], axis=0)[:, None]
  mean = s / jnp.maximum(cnt, 1.0)
  acc = (jnp.dot(mean, wl_ref[...], preferred_element_type=jnp.float32)
         + jnp.dot(x_ref[...], wr_ref[...], preferred_element_type=jnp.float32)
         + b_ref[...])
  o_ref[...] = jnp.maximum(acc, 0.0) if relu else acc


def _tc_layer(p, c, x, wlt, wrt, b, relu):
  return pl.pallas_call(
      functools.partial(_tc_layer_body, relu),
      grid=(NP // BN,),
      in_specs=[
          pl.BlockSpec((NC, BN, D), lambda i: (0, i, 0)),
          pl.BlockSpec((NW, BN), lambda i: (0, i)),
          pl.BlockSpec((BN, D), lambda i: (i, 0)),
          pl.BlockSpec((D, D), lambda i: (0, 0)),
          pl.BlockSpec((D, D), lambda i: (0, 0)),
          pl.BlockSpec((1, D), lambda i: (0, 0)),
      ],
      out_specs=pl.BlockSpec((BN, D), lambda i: (i, 0)),
      out_shape=jax.ShapeDtypeStruct((NP, D), jnp.float32),
  )(p, c, x, wlt, wrt, b)


def kernel(edge_index, emb, W1l, b1l, W1r, W2l, b2l, W2r):
  src = edge_index[0]
  dst = edge_index[1]
  pad = EPAD - E
  srcs = jnp.concatenate(
      [src, jnp.zeros((pad,), jnp.int32)]).reshape(TOTCH, CH)
  pad_dst = N + jnp.arange(pad, dtype=jnp.int32) % (NP - N)
  dsts = jnp.concatenate([dst, pad_dst]).reshape(TOTCH, CH)
  embp = jnp.pad(emb, ((0, NP - N), (0, 0)))

  p1, c = _sc_aggregate(embp, srcs, dsts, with_counts=True)
  h = _tc_layer(p1, c, embp, W1l.T, W1r.T, b1l.reshape(1, D), relu=True)
  (p2,) = _sc_aggregate(h, srcs, dsts, with_counts=False)
  out = _tc_layer(p2, c, h, W2l.T, W2r.T, b2l.reshape(1, D), relu=False)
  return out[:N]
